# baseline (speedup 1.0000x reference)
import functools

import jax
import jax.numpy as jnp
from jax import lax
from jax.experimental import pallas as pl
from jax.experimental.pallas import tpu as pltpu
from jax.experimental.pallas import tpu_sc as plsc

N_NODES = 10000
N_EDGES = 320000
D = 128

NC = 2
NS = 16
CH = 40
NBUF = 4
NCHUNK_A = 308
NCHUNK_B = 192
EPT_A = NCHUNK_A * CH
EPT_B = NCHUNK_B * CH
REGION_A = NS * EPT_A
N_PAD = 10240
ROWS_PT = N_PAD // NS
LANES = 16


def _sc_segment_sum(feature, src1, dst1):
    mesh = plsc.VectorSubcoreMesh(core_axis_name="c", subcore_axis_name="s")

    @functools.partial(
        pl.kernel,
        mesh=mesh,
        out_type=jax.ShapeDtypeStruct((NC, N_PAD, D), jnp.float32),
        scratch_types=[
            pltpu.VMEM((EPT_A,), jnp.int32),
            pltpu.VMEM((EPT_A,), jnp.int32),
        ] + [pltpu.VMEM((CH, D), jnp.float32)] * NBUF
          + [pltpu.VMEM_SHARED((N_PAD, D), jnp.float32)]
          + [pltpu.SemaphoreType.DMA] * NBUF,
    )
    def k(feat_hbm, src_hbm, dst_hbm, out_hbm, sidx_v, didx_v, *rest):
        rbufs = rest[:NBUF]
        acc_sh = rest[NBUF]
        sems = rest[NBUF + 1:]
        rows0 = rbufs[0]
        cid = lax.axis_index("c")
        sid = lax.axis_index("s")

        zero = jnp.zeros((LANES,), jnp.float32)

        def zbody(i, carry):
            r = i // (D // LANES)
            col = (i % (D // LANES)) * LANES
            rows0[r, pl.ds(col, LANES)] = zero
            return carry

        lax.fori_loop(0, CH * (D // LANES), zbody, 0)

        row0 = sid * ROWS_PT

        def zcopy(j, carry):
            pltpu.sync_copy(rows0, acc_sh.at[pl.ds(row0 + j * CH, CH)])
            return carry

        lax.fori_loop(0, ROWS_PT // CH, zcopy, 0)

        def gather(c, buf, sem):
            pltpu.async_copy(
                feat_hbm.at[sidx_v.at[pl.ds(c * CH, CH)]], buf, sem)

        def gwait(buf, sem):
            pltpu.make_async_copy(
                feat_hbm.at[sidx_v.at[pl.ds(0, CH)]], buf, sem).wait()

        def scat(c, buf):
            pltpu.sync_copy(
                buf, acc_sh.at[didx_v.at[pl.ds(c * CH, CH)]], add=True)

        bufs = tuple(zip(rbufs, sems))

        def run_edges(ebase, ept, nchunk):
            pltpu.sync_copy(src_hbm.at[pl.ds(ebase, ept)],
                            sidx_v.at[pl.ds(0, ept)])
            pltpu.sync_copy(dst_hbm.at[pl.ds(ebase, ept)],
                            didx_v.at[pl.ds(0, ept)])
            for t in range(NBUF - 1):
                gather(t, rbufs[t], sems[t])

            def ebody(i, carry):
                c = NBUF * i
                for t in range(NBUF):
                    buf, sem = bufs[t]
                    nbuf, nsem = bufs[(t + NBUF - 1) % NBUF]
                    cn = jnp.where(c + t + NBUF - 1 < nchunk,
                                   c + t + NBUF - 1, 0)
                    gwait(buf, sem)
                    gather(cn, nbuf, nsem)
                    scat(c + t, buf)
                return carry

            lax.fori_loop(0, nchunk // NBUF, ebody, 0)
            for t in range(NBUF - 1):
                gwait(rbufs[t], sems[t])

        plsc.subcore_barrier()

        @pl.when(cid == 0)
        def _():
            run_edges(sid * EPT_A, EPT_A, NCHUNK_A)

        @pl.when(cid == 1)
        def _():
            run_edges(REGION_A + sid * EPT_B, EPT_B, NCHUNK_B)

        plsc.subcore_barrier()

        for j in range(ROWS_PT // CH):
            buf, sem = bufs[j % NBUF]
            r = row0 + j * CH
            if j >= NBUF:
                pltpu.make_async_copy(buf, out_hbm.at[cid, pl.ds(r, CH)],
                                      sem).wait()
            pltpu.sync_copy(acc_sh.at[pl.ds(r, CH)], buf)
            pltpu.async_copy(buf, out_hbm.at[cid, pl.ds(r, CH)], sem)
        for buf, sem in bufs:
            pltpu.make_async_copy(buf, out_hbm.at[cid, pl.ds(row0, CH)],
                                  sem).wait()

    return k(feature, src1, dst1)


def _tc_linear(partials, wt, bias):
    RB = 2000

    def mm(p_ref, w_ref, b_ref, o_ref):
        acc = p_ref[0] + p_ref[1]
        o_ref[...] = (
            jnp.dot(acc, w_ref[...], preferred_element_type=jnp.float32)
            + b_ref[...]
        )

    return pl.pallas_call(
        mm,
        grid=(N_NODES // RB,),
        in_specs=[
            pl.BlockSpec((NC, RB, D), lambda i: (0, i, 0)),
            pl.BlockSpec((D, D), lambda i: (0, 0)),
            pl.BlockSpec((1, D), lambda i: (0, 0)),
        ],
        out_specs=pl.BlockSpec((RB, D), lambda i: (i, 0)),
        out_shape=jax.ShapeDtypeStruct((N_NODES, D), jnp.float32),
    )(partials, wt, bias.reshape(1, D))


def kernel(feature, edge_index, W, b):
    ei = edge_index.astype(jnp.int32)
    partials = _sc_segment_sum(feature, ei[0], ei[1])
    return _tc_linear(partials, W.T, b)


# speedup vs baseline: 1.0023x; 1.0023x over previous
import functools

import jax
import jax.numpy as jnp
from jax import lax
from jax.experimental import pallas as pl
from jax.experimental.pallas import tpu as pltpu
from jax.experimental.pallas import tpu_sc as plsc

N_NODES = 10000
N_EDGES = 320000
D = 128

NC = 2
NS = 16
CH = 40
NBUF = 4
NCHUNK_A = 308
NCHUNK_B = 192
EPT_A = NCHUNK_A * CH
EPT_B = NCHUNK_B * CH
REGION_A = NS * EPT_A
N_PAD = 10240
ROWS_PT = N_PAD // NS
LANES = 16


def _sc_segment_sum(feature, src1, dst1):
    mesh = plsc.VectorSubcoreMesh(core_axis_name="c", subcore_axis_name="s")

    @functools.partial(
        pl.kernel,
        mesh=mesh,
        out_type=jax.ShapeDtypeStruct((NC, N_PAD, D), jnp.float32),
        scratch_types=[
            pltpu.VMEM((EPT_A,), jnp.int32),
            pltpu.VMEM((EPT_A,), jnp.int32),
        ] + [pltpu.VMEM((CH, D), jnp.float32)] * NBUF
          + [pltpu.VMEM_SHARED((N_PAD, D), jnp.float32)]
          + [pltpu.SemaphoreType.DMA] * NBUF,
    )
    def k(feat_hbm, src_hbm, dst_hbm, out_hbm, sidx_v, didx_v, *rest):
        rbufs = rest[:NBUF]
        acc_sh = rest[NBUF]
        sems = rest[NBUF + 1:]
        rows0 = rbufs[0]
        cid = lax.axis_index("c")
        sid = lax.axis_index("s")

        zero = jnp.zeros((LANES,), jnp.float32)

        def zbody(i, carry):
            r = i // (D // LANES)
            col = (i % (D // LANES)) * LANES
            rows0[r, pl.ds(col, LANES)] = zero
            return carry

        lax.fori_loop(0, CH * (D // LANES), zbody, 0)

        row0 = sid * ROWS_PT

        def zcopy(j, carry):
            pltpu.sync_copy(rows0, acc_sh.at[pl.ds(row0 + j * CH, CH)])
            return carry

        lax.fori_loop(0, ROWS_PT // CH, zcopy, 0)

        def gather(c, buf, sem):
            pltpu.async_copy(
                feat_hbm.at[sidx_v.at[pl.ds(c * CH, CH)]], buf, sem)

        def gwait(buf, sem):
            pltpu.make_async_copy(
                feat_hbm.at[sidx_v.at[pl.ds(0, CH)]], buf, sem).wait()

        def scat(c, buf):
            pltpu.sync_copy(
                buf, acc_sh.at[didx_v.at[pl.ds(c * CH, CH)]], add=True)

        bufs = tuple(zip(rbufs, sems))

        def run_edges(ebase, ept, nchunk):
            pltpu.sync_copy(src_hbm.at[pl.ds(ebase, ept)],
                            sidx_v.at[pl.ds(0, ept)])
            pltpu.sync_copy(dst_hbm.at[pl.ds(ebase, ept)],
                            didx_v.at[pl.ds(0, ept)])
            for t in range(NBUF - 1):
                gather(t, rbufs[t], sems[t])

            def ebody(i, carry):
                c = NBUF * i
                for t in range(NBUF):
                    buf, sem = bufs[t]
                    nbuf, nsem = bufs[(t + NBUF - 1) % NBUF]
                    cn = jnp.where(c + t + NBUF - 1 < nchunk,
                                   c + t + NBUF - 1, 0)
                    gwait(buf, sem)
                    gather(cn, nbuf, nsem)
                    scat(c + t, buf)
                return carry

            lax.fori_loop(0, nchunk // NBUF, ebody, 0)
            for t in range(NBUF - 1):
                gwait(rbufs[t], sems[t])

        plsc.subcore_barrier()

        @pl.when(cid == 1)
        def _():
            run_edges(sid * EPT_A, EPT_A, NCHUNK_A)

        @pl.when(cid == 0)
        def _():
            run_edges(REGION_A + sid * EPT_B, EPT_B, NCHUNK_B)

        plsc.subcore_barrier()

        for j in range(ROWS_PT // CH):
            buf, sem = bufs[j % NBUF]
            r = row0 + j * CH
            if j >= NBUF:
                pltpu.make_async_copy(buf, out_hbm.at[cid, pl.ds(r, CH)],
                                      sem).wait()
            pltpu.sync_copy(acc_sh.at[pl.ds(r, CH)], buf)
            pltpu.async_copy(buf, out_hbm.at[cid, pl.ds(r, CH)], sem)
        for buf, sem in bufs:
            pltpu.make_async_copy(buf, out_hbm.at[cid, pl.ds(row0, CH)],
                                  sem).wait()

    return k(feature, src1, dst1)


def _tc_linear(partials, wt, bias):
    RB = 2000

    def mm(p_ref, w_ref, b_ref, o_ref):
        acc = p_ref[0] + p_ref[1]
        o_ref[...] = (
            jnp.dot(acc, w_ref[...], preferred_element_type=jnp.float32)
            + b_ref[...]
        )

    return pl.pallas_call(
        mm,
        grid=(N_NODES // RB,),
        in_specs=[
            pl.BlockSpec((NC, RB, D), lambda i: (0, i, 0)),
            pl.BlockSpec((D, D), lambda i: (0, 0)),
            pl.BlockSpec((1, D), lambda i: (0, 0)),
        ],
        out_specs=pl.BlockSpec((RB, D), lambda i: (i, 0)),
        out_shape=jax.ShapeDtypeStruct((N_NODES, D), jnp.float32),
    )(partials, wt, bias.reshape(1, D))


def kernel(feature, edge_index, W, b):
    ei = edge_index.astype(jnp.int32)
    partials = _sc_segment_sum(feature, ei[0], ei[1])
    return _tc_linear(partials, W.T, b)


# speedup vs baseline: 1.1895x; 1.1867x over previous
import functools

import jax
import jax.numpy as jnp
from jax import lax
from jax.experimental import pallas as pl
from jax.experimental.pallas import tpu as pltpu
from jax.experimental.pallas import tpu_sc as plsc

N_NODES = 10000
N_EDGES = 320000
D = 128

NC = 2
NS = 16
NW = NC * NS
EPT = N_EDGES // NW
CH = 40
NCHUNK = EPT // CH
NBUF = 5
N_PAD = 10240
ROWS_PT = N_PAD // NS
LANES = 16


def _sc_segment_sum(feature, src2, dst2):
    mesh = plsc.VectorSubcoreMesh(core_axis_name="c", subcore_axis_name="s")

    @functools.partial(
        pl.kernel,
        mesh=mesh,
        out_type=jax.ShapeDtypeStruct((NC, N_PAD, D), jnp.float32),
        scratch_types=[
            pltpu.VMEM((EPT,), jnp.int32),
            pltpu.VMEM((EPT,), jnp.int32),
        ] + [pltpu.VMEM((CH, D), jnp.float32)] * NBUF
          + [pltpu.VMEM_SHARED((N_PAD, D), jnp.float32)]
          + [pltpu.SemaphoreType.DMA] * NBUF,
    )
    def k(feat_hbm, src_hbm, dst_hbm, out_hbm, sidx_v, didx_v, *rest):
        rbufs = rest[:NBUF]
        acc_sh = rest[NBUF]
        sems = rest[NBUF + 1:]
        rows0 = rbufs[0]
        cid = lax.axis_index("c")
        sid = lax.axis_index("s")
        wid = sid * NC + cid

        zero = jnp.zeros((LANES,), jnp.float32)

        def zbody(i, carry):
            r = i // (D // LANES)
            col = (i % (D // LANES)) * LANES
            rows0[r, pl.ds(col, LANES)] = zero
            return carry

        lax.fori_loop(0, CH * (D // LANES), zbody, 0)

        row0 = sid * ROWS_PT

        def zcopy(j, carry):
            pltpu.sync_copy(rows0, acc_sh.at[pl.ds(row0 + j * CH, CH)])
            return carry

        lax.fori_loop(0, ROWS_PT // CH, zcopy, 0)

        pltpu.sync_copy(src_hbm.at[wid], sidx_v)
        pltpu.sync_copy(dst_hbm.at[wid], didx_v)
        plsc.subcore_barrier()

        def gather(c, buf, sem):
            pltpu.async_copy(
                feat_hbm.at[sidx_v.at[pl.ds(c * CH, CH)]], buf, sem)

        def gwait(buf, sem):
            pltpu.make_async_copy(
                feat_hbm.at[sidx_v.at[pl.ds(0, CH)]], buf, sem).wait()

        def scat(c, buf):
            pltpu.sync_copy(
                buf, acc_sh.at[didx_v.at[pl.ds(c * CH, CH)]], add=True)

        bufs = tuple(zip(rbufs, sems))
        for t in range(NBUF - 1):
            gather(t, rbufs[t], sems[t])

        def ebody(i, carry):
            c = NBUF * i
            for t in range(NBUF):
                buf, sem = bufs[t]
                nbuf, nsem = bufs[(t + NBUF - 1) % NBUF]
                cn = jnp.where(c + t + NBUF - 1 < NCHUNK, c + t + NBUF - 1, 0)
                gwait(buf, sem)
                gather(cn, nbuf, nsem)
                scat(c + t, buf)
            return carry

        lax.fori_loop(0, NCHUNK // NBUF, ebody, 0)
        for t in range(NBUF - 1):
            gwait(rbufs[t], sems[t])
        plsc.subcore_barrier()

        for j in range(ROWS_PT // CH):
            buf, sem = bufs[j % NBUF]
            r = row0 + j * CH
            if j >= NBUF:
                pltpu.make_async_copy(buf, out_hbm.at[cid, pl.ds(r, CH)],
                                      sem).wait()
            pltpu.sync_copy(acc_sh.at[pl.ds(r, CH)], buf)
            pltpu.async_copy(buf, out_hbm.at[cid, pl.ds(r, CH)], sem)
        for buf, sem in bufs:
            pltpu.make_async_copy(buf, out_hbm.at[cid, pl.ds(row0, CH)],
                                  sem).wait()

    return k(feature, src2, dst2)


def _tc_linear(partials, wt, bias):
    RB = 2000

    def mm(p_ref, w_ref, b_ref, o_ref):
        acc = p_ref[0] + p_ref[1]
        o_ref[...] = (
            jnp.dot(acc, w_ref[...], preferred_element_type=jnp.float32)
            + b_ref[...]
        )

    return pl.pallas_call(
        mm,
        grid=(N_NODES // RB,),
        in_specs=[
            pl.BlockSpec((NC, RB, D), lambda i: (0, i, 0)),
            pl.BlockSpec((D, D), lambda i: (0, 0)),
            pl.BlockSpec((1, D), lambda i: (0, 0)),
        ],
        out_specs=pl.BlockSpec((RB, D), lambda i: (i, 0)),
        out_shape=jax.ShapeDtypeStruct((N_NODES, D), jnp.float32),
    )(partials, wt, bias.reshape(1, D))


def kernel(feature, edge_index, W, b):
    ei = edge_index.astype(jnp.int32)
    src2 = ei[0].reshape(NW, EPT)
    dst2 = ei[1].reshape(NW, EPT)
    partials = _sc_segment_sum(feature, src2, dst2)
    return _tc_linear(partials, W.T, b)
